# Initial kernel scaffold; baseline (speedup 1.0000x reference)
import jax
import jax.numpy as jnp
from jax.experimental import pallas as pl


def kernel(categories, table):
    raise NotImplementedError("write your pallas kernel here")


# speedup vs baseline: 1.0329x; 1.0329x over previous
import functools

import jax
import jax.numpy as jnp
from jax import lax
from jax.experimental import pallas as pl
from jax.experimental.pallas import tpu as pltpu
from jax.experimental.pallas import tpu_sc as plsc

NUM_CORES = 2
NUM_SUBCORES = 16
NUM_WORKERS = NUM_CORES * NUM_SUBCORES
LANES = 16

EMBED_DIM = 32
IDX_W = 128
K = 4
CHUNK = K * IDX_W


def _sc_body(rows_per_worker, cat_hbm, table_hbm, out_hbm, idx_v, rows_v, sems):
    wid = lax.axis_index("s") * NUM_CORES + lax.axis_index("c")
    row_base = wid * rows_per_worker
    n_chunks = rows_per_worker // K

    def start_gathers(g, buf):
        pltpu.sync_copy(cat_hbm.at[pl.ds(row_base + g * K, K)], idx_v.at[buf])
        for j in range(K):
            pltpu.async_copy(
                table_hbm.at[idx_v.at[buf, j]],
                rows_v.at[buf, pl.ds(j * IDX_W, IDX_W)],
                sems.at[buf],
            )

    def wait_gathers(buf):
        for j in range(K):
            pltpu.make_async_copy(
                table_hbm.at[idx_v.at[buf, j]],
                rows_v.at[buf, pl.ds(j * IDX_W, IDX_W)],
                sems.at[buf],
            ).wait()

    start_gathers(0, 0)

    def chunk_body(g, _):
        cur = lax.rem(g, 2)
        nxt = lax.rem(g + 1, 2)

        @pl.when(g + 1 < n_chunks)
        def _():
            start_gathers(g + 1, nxt)

        wait_gathers(cur)

        @pl.loop(0, CHUNK, unroll=8)
        def _(i):
            for h in range(EMBED_DIM // LANES):
                sl = pl.ds(h * LANES, LANES)
                rows_v[cur, i, sl] = jnp.maximum(rows_v[cur, i, sl], 0.0)

        pltpu.sync_copy(
            rows_v.at[cur],
            out_hbm.at[pl.ds((row_base + g * K) * IDX_W, CHUNK)],
        )

    lax.fori_loop(0, n_chunks, chunk_body, None)


@jax.jit
def kernel(categories, table):
    batch, hist = categories.shape
    n_total = batch * hist
    rows_per_worker = n_total // (NUM_WORKERS * IDX_W)
    idx2d = categories.reshape(n_total // IDX_W, IDX_W).astype(jnp.int32)

    mesh = plsc.VectorSubcoreMesh(
        core_axis_name="c", subcore_axis_name="s",
        num_cores=NUM_CORES, num_subcores=NUM_SUBCORES,
    )
    out = pl.kernel(
        functools.partial(_sc_body, rows_per_worker),
        out_type=jax.ShapeDtypeStruct((n_total, EMBED_DIM), jnp.float32),
        mesh=mesh,
        compiler_params=pltpu.CompilerParams(use_tc_tiling_on_sc=False),
        scratch_types=[
            pltpu.VMEM((2, K, IDX_W), jnp.int32),
            pltpu.VMEM((2, CHUNK, EMBED_DIM), jnp.float32),
            pltpu.SemaphoreType.DMA((2,)),
        ],
    )(idx2d, table)
    return out.reshape(batch, hist, EMBED_DIM)


# speedup vs baseline: 1.3638x; 1.3204x over previous
import functools

import jax
import jax.numpy as jnp
from jax import lax
from jax.experimental import pallas as pl
from jax.experimental.pallas import tpu as pltpu
from jax.experimental.pallas import tpu_sc as plsc

NUM_CORES = 2
NUM_SUBCORES = 16
NUM_WORKERS = NUM_CORES * NUM_SUBCORES
LANES = 16

EMBED_DIM = 32
HIST = 50
ROWS_PER_CHUNK = 8
CHUNK = ROWS_PER_CHUNK * HIST
GATHER_SPLITS = (128, 128, 128, 16)


def _sc_body(rows_per_worker, cat_hbm, table_hbm, out_hbm, idx_v, rows_v, out3_v,
             sems, out_sems):
    wid = lax.axis_index("s") * NUM_CORES + lax.axis_index("c")
    row_base = wid * rows_per_worker
    n_chunks = rows_per_worker // ROWS_PER_CHUNK

    def stage_and_gather(g, buf):
        flat0 = (row_base + g * ROWS_PER_CHUNK) * HIST
        off = 0
        for j, w in enumerate(GATHER_SPLITS):
            pltpu.sync_copy(cat_hbm.at[pl.ds(flat0 + off, w)],
                            idx_v.at[buf, j, pl.ds(0, w)])
            off += w
        off = 0
        for j, w in enumerate(GATHER_SPLITS):
            pltpu.async_copy(
                table_hbm.at[idx_v.at[buf, j, pl.ds(0, w)]],
                rows_v.at[buf, pl.ds(off, w)],
                sems.at[buf],
            )
            off += w

    def wait_gathers(buf):
        off = 0
        for j, w in enumerate(GATHER_SPLITS):
            pltpu.make_async_copy(
                table_hbm.at[idx_v.at[buf, j, pl.ds(0, w)]],
                rows_v.at[buf, pl.ds(off, w)],
                sems.at[buf],
            ).wait()
            off += w

    def wait_out(buf):
        pltpu.make_async_copy(
            out3_v.at[buf],
            out_hbm.at[pl.ds(0, ROWS_PER_CHUNK)],
            out_sems.at[buf],
        ).wait()

    stage_and_gather(0, 0)

    def chunk_body(g, _):
        cur = lax.rem(g, 2)
        nxt = lax.rem(g + 1, 2)

        @pl.when(g + 1 < n_chunks)
        def _():
            stage_and_gather(g + 1, nxt)

        wait_gathers(cur)

        @pl.when(g >= 2)
        def _():
            wait_out(cur)

        @pl.loop(0, ROWS_PER_CHUNK)
        def _(bi):
            @pl.loop(0, HIST, unroll=5)
            def _(li):
                i = bi * HIST + li
                for h in range(EMBED_DIM // LANES):
                    sl = pl.ds(h * LANES, LANES)
                    out3_v[cur, bi, li, sl] = jnp.maximum(rows_v[cur, i, sl], 0.0)

        pltpu.async_copy(
            out3_v.at[cur],
            out_hbm.at[pl.ds(row_base + g * ROWS_PER_CHUNK, ROWS_PER_CHUNK)],
            out_sems.at[cur],
        )

    lax.fori_loop(0, n_chunks, chunk_body, None)

    wait_out(lax.rem(n_chunks, 2))
    wait_out(lax.rem(n_chunks + 1, 2))


@jax.jit
def kernel(categories, table):
    batch, hist = categories.shape
    rows_per_worker = batch // NUM_WORKERS
    flat_idx = categories.reshape(batch * hist).astype(jnp.int32)

    mesh = plsc.VectorSubcoreMesh(
        core_axis_name="c", subcore_axis_name="s",
        num_cores=NUM_CORES, num_subcores=NUM_SUBCORES,
    )
    out = pl.kernel(
        functools.partial(_sc_body, rows_per_worker),
        out_type=jax.ShapeDtypeStruct((batch, hist, EMBED_DIM), jnp.float32),
        mesh=mesh,
        compiler_params=pltpu.CompilerParams(use_tc_tiling_on_sc=False),
        scratch_types=[
            pltpu.VMEM((2, len(GATHER_SPLITS), 128), jnp.int32),
            pltpu.VMEM((2, CHUNK, EMBED_DIM), jnp.float32),
            pltpu.VMEM((2, ROWS_PER_CHUNK, HIST, EMBED_DIM), jnp.float32),
            pltpu.SemaphoreType.DMA((2,)),
            pltpu.SemaphoreType.DMA((2,)),
        ],
    )(flat_idx, table)
    return out


# speedup vs baseline: 1.5012x; 1.1008x over previous
import functools

import jax
import jax.numpy as jnp
from jax import lax
from jax.experimental import pallas as pl
from jax.experimental.pallas import tpu as pltpu
from jax.experimental.pallas import tpu_sc as plsc

NUM_CORES = 2
NUM_SUBCORES = 16
NUM_WORKERS = NUM_CORES * NUM_SUBCORES
LANES = 16

EMBED_DIM = 32
HIST = 50
ROWS_PER_CHUNK = 8
CHUNK = ROWS_PER_CHUNK * HIST
GATHER_SPLITS = (128, 128, 128, 16)


def _sc_body(rows_per_worker, cat_hbm, table_hbm, out_hbm, idx_v, rows_v, out3_v,
             sems, out_sems):
    wid = lax.axis_index("s") * NUM_CORES + lax.axis_index("c")
    row_base = wid * rows_per_worker
    n_chunks = rows_per_worker // ROWS_PER_CHUNK

    pltpu.sync_copy(cat_hbm.at[pl.ds(row_base * HIST, rows_per_worker * HIST)],
                    idx_v)

    def start_gathers(g, buf):
        off = 0
        for w in GATHER_SPLITS:
            pltpu.async_copy(
                table_hbm.at[idx_v.at[pl.ds(g * CHUNK + off, w)]],
                rows_v.at[buf, pl.ds(off, w)],
                sems.at[buf],
            )
            off += w

    def wait_gathers(g, buf):
        off = 0
        for w in GATHER_SPLITS:
            pltpu.make_async_copy(
                table_hbm.at[idx_v.at[pl.ds(g * CHUNK + off, w)]],
                rows_v.at[buf, pl.ds(off, w)],
                sems.at[buf],
            ).wait()
            off += w

    def wait_out(buf):
        pltpu.make_async_copy(
            out3_v.at[buf],
            out_hbm.at[pl.ds(0, ROWS_PER_CHUNK)],
            out_sems.at[buf],
        ).wait()

    start_gathers(0, 0)

    def chunk_body(g, _):
        cur = lax.rem(g, 2)
        nxt = lax.rem(g + 1, 2)

        @pl.when(g + 1 < n_chunks)
        def _():
            start_gathers(g + 1, nxt)

        wait_gathers(g, cur)

        @pl.when(g >= 2)
        def _():
            wait_out(cur)

        @pl.loop(0, ROWS_PER_CHUNK)
        def _(bi):
            @pl.loop(0, HIST, unroll=5)
            def _(li):
                i = bi * HIST + li
                for h in range(EMBED_DIM // LANES):
                    sl = pl.ds(h * LANES, LANES)
                    out3_v[cur, bi, li, sl] = jnp.maximum(rows_v[cur, i, sl], 0.0)

        pltpu.async_copy(
            out3_v.at[cur],
            out_hbm.at[pl.ds(row_base + g * ROWS_PER_CHUNK, ROWS_PER_CHUNK)],
            out_sems.at[cur],
        )

    lax.fori_loop(0, n_chunks, chunk_body, None)

    wait_out(lax.rem(n_chunks, 2))
    wait_out(lax.rem(n_chunks + 1, 2))


@jax.jit
def kernel(categories, table):
    batch, hist = categories.shape
    rows_per_worker = batch // NUM_WORKERS
    flat_idx = categories.reshape(batch * hist).astype(jnp.int32)

    mesh = plsc.VectorSubcoreMesh(
        core_axis_name="c", subcore_axis_name="s",
        num_cores=NUM_CORES, num_subcores=NUM_SUBCORES,
    )
    out = pl.kernel(
        functools.partial(_sc_body, rows_per_worker),
        out_type=jax.ShapeDtypeStruct((batch, hist, EMBED_DIM), jnp.float32),
        mesh=mesh,
        compiler_params=pltpu.CompilerParams(use_tc_tiling_on_sc=False),
        scratch_types=[
            pltpu.VMEM((rows_per_worker * HIST,), jnp.int32),
            pltpu.VMEM((2, CHUNK, EMBED_DIM), jnp.float32),
            pltpu.VMEM((2, ROWS_PER_CHUNK, HIST, EMBED_DIM), jnp.float32),
            pltpu.SemaphoreType.DMA((2,)),
            pltpu.SemaphoreType.DMA((2,)),
        ],
    )(flat_idx, table)
    return out
